# baseline (speedup 1.0000x reference)
import functools

import jax
import jax.numpy as jnp
from jax.experimental import pallas as pl

_F32 = jnp.float32
_BF16 = jnp.bfloat16


def _fps_body(xt_ref, out_ref, *, n, npoint):
    b = xt_ref.shape[1]
    x0 = xt_ref[0]
    x1 = xt_ref[1]
    x2 = xt_ref[2]
    iota = jax.lax.broadcasted_iota(jnp.int32, (b, n), 1)

    def body(i, state):
        dists, far = state
        onehot = iota == far
        c0 = jnp.sum(jnp.where(onehot, x0, 0.0), axis=1, keepdims=True)
        c1 = jnp.sum(jnp.where(onehot, x1, 0.0), axis=1, keepdims=True)
        c2 = jnp.sum(jnp.where(onehot, x2, 0.0), axis=1, keepdims=True)
        cent = jnp.concatenate([c0, c1, c2], axis=1).reshape(b, 1, 3)
        out_ref[:, pl.ds(i, 1), :] = cent
        d = (x0 - c0) ** 2 + (x1 - c1) ** 2 + (x2 - c2) ** 2
        dists = jnp.minimum(dists, d)
        m = jnp.max(dists, axis=1, keepdims=True)
        cand = jnp.where(dists == m, iota, n)
        far = jnp.min(cand, axis=1, keepdims=True)
        return dists, far

    dists = jnp.full((b, n), 1e10, _F32)
    far = jnp.zeros((b, 1), jnp.int32)
    jax.lax.fori_loop(0, npoint, body, (dists, far))


def _fps(xt, npoint):
    _, b, n = xt.shape
    return pl.pallas_call(
        functools.partial(_fps_body, n=n, npoint=npoint),
        out_shape=jax.ShapeDtypeStruct((b, npoint, 3), _F32),
    )(xt)


def _sa_body(cat_ref, xt_ref, nx_ref, w0t_ref, b0_ref, w1t_ref, b1_ref,
             out_ref, *, r2, nsample):
    tri = None
    for t in range(cat_ref.shape[0]):
        tri = _sa_one(cat_ref[t], xt_ref[t], nx_ref[t], w0t_ref, b0_ref,
                      w1t_ref, b1_ref, out_ref, t, tri, r2=r2,
                      nsample=nsample)


def _sa_one(cat, xt, nx, w0t_ref, b0_ref, w1t_ref, b1_ref, out_ref, t, tri,
            *, r2, nsample):
    n = cat.shape[0]
    s = nx.shape[0]

    p = jnp.dot(cat, w0t_ref[...], preferred_element_type=_F32)
    off = jnp.dot(nx, w0t_ref[0:3, :], preferred_element_type=_F32)

    sqr = ((nx[:, 0:1] - xt[0:1, :]) ** 2
           + (nx[:, 1:2] - xt[1:2, :]) ** 2
           + (nx[:, 2:3] - xt[2:3, :]) ** 2)
    mask = (sqr < r2).astype(_F32)

    if tri is None:
        tri = (jax.lax.broadcasted_iota(jnp.int32, (n, n), 0)
               <= jax.lax.broadcasted_iota(jnp.int32, (n, n), 1)).astype(_BF16)
    rank = jnp.dot(mask.astype(_BF16), tri, preferred_element_type=_F32)
    count = rank[:, n - 1:n]
    vrank = jnp.where(mask > 0.5, rank, 0.0)

    b0 = b0_ref[...]
    b1 = b1_ref[...]
    w1t = w1t_ref[...].astype(_BF16)
    pb = p.astype(_BF16)
    cm = pb.shape[1]

    kvec = (jax.lax.broadcasted_iota(jnp.int32, (nsample, 1, 1), 0)
            .astype(_F32) + 1.0)
    sel = (vrank[None, :, :] == kvec).astype(_BF16)
    g = jnp.dot(sel.reshape(nsample * s, n), pb,
                preferred_element_type=_F32).reshape(nsample, s, cm)
    g0 = g[0]
    g = jnp.where(count[None, :, :] >= kvec, g, g0[None, :, :])
    a = jnp.maximum(g - off[None, :, :] + b0[None, :, :], 0.0)
    z = jnp.dot(a.astype(_BF16).reshape(nsample * s, cm), w1t,
                preferred_element_type=_F32)
    z = z.reshape(nsample, s, z.shape[-1]) + b1[None, :, :]
    out_ref[t] = jnp.maximum(jnp.max(z, axis=0), 0.0)
    return tri


def _sa(cat, xt, nx, w0, b0, w1, b1, radius, nsample):
    b, n, c = cat.shape
    s = nx.shape[1]
    cm = w0.shape[0]
    co = w1.shape[0]
    bpg = 8
    grid = (b // bpg,)
    return pl.pallas_call(
        functools.partial(_sa_body, r2=radius * radius, nsample=nsample),
        grid=grid,
        in_specs=[
            pl.BlockSpec((bpg, n, c), lambda i: (i, 0, 0)),
            pl.BlockSpec((bpg, 3, n), lambda i: (i, 0, 0)),
            pl.BlockSpec((bpg, s, 3), lambda i: (i, 0, 0)),
            pl.BlockSpec((c, cm), lambda i: (0, 0)),
            pl.BlockSpec((1, cm), lambda i: (0, 0)),
            pl.BlockSpec((cm, co), lambda i: (0, 0)),
            pl.BlockSpec((1, co), lambda i: (0, 0)),
        ],
        out_specs=pl.BlockSpec((bpg, s, co), lambda i: (i, 0, 0)),
        out_shape=jax.ShapeDtypeStruct((b, s, co), _F32),
    )(cat, xt, nx, w0.T, b0[None, :], w1.T, b1[None, :])


def _tail_body(cat_ref, w20t_ref, b20_ref, w21t_ref, b21_ref,
               cw0t_ref, cb0_ref, cw1t_ref, cb1_ref, cw2t_ref, cb2_ref,
               rw0t_ref, rb0_ref, rw1t_ref, rb1_ref, rw2t_ref, rb2_ref,
               box_ref, cls_ref):
    b, ns, c = cat_ref.shape
    h = cat_ref[...].reshape(b * ns, c)
    h = jnp.maximum(jnp.dot(h, w20t_ref[...], preferred_element_type=_F32)
                    + b20_ref[...], 0.0)
    h = jnp.maximum(jnp.dot(h, w21t_ref[...], preferred_element_type=_F32)
                    + b21_ref[...], 0.0)
    g = jnp.max(h.reshape(b, ns, h.shape[-1]), axis=1)

    cc = jnp.maximum(jnp.dot(g, cw0t_ref[...], preferred_element_type=_F32)
                     + cb0_ref[...], 0.0)
    cc = jnp.maximum(jnp.dot(cc, cw1t_ref[...], preferred_element_type=_F32)
                     + cb1_ref[...], 0.0)
    cc = jnp.dot(cc, cw2t_ref[...], preferred_element_type=_F32) + cb2_ref[...]
    cls_ref[...] = jax.nn.sigmoid(cc)

    rr = jnp.maximum(jnp.dot(g, rw0t_ref[...], preferred_element_type=_F32)
                     + rb0_ref[...], 0.0)
    rr = jnp.maximum(jnp.dot(rr, rw1t_ref[...], preferred_element_type=_F32)
                     + rb1_ref[...], 0.0)
    box_ref[...] = (jnp.dot(rr, rw2t_ref[...], preferred_element_type=_F32)
                    + rb2_ref[...])


def _tail(cat2, w20, b20, w21, b21, cw0, cb0, cw1, cb1, cw2, cb2,
          rw0, rb0, rw1, rb1, rw2, rb2):
    b = cat2.shape[0]
    args = (cat2, w20.T, b20[None, :], w21.T, b21[None, :],
            cw0.T, cb0[None, :], cw1.T, cb1[None, :], cw2.T, cb2[None, :],
            rw0.T, rb0[None, :], rw1.T, rb1[None, :], rw2.T, rb2[None, :])
    return pl.pallas_call(
        _tail_body,
        out_shape=(jax.ShapeDtypeStruct((b, 7), _F32),
                   jax.ShapeDtypeStruct((b, 1), _F32)),
    )(*args)


def kernel(x, sa0_w0, sa0_b0, sa0_w1, sa0_b1, sa1_w0, sa1_b0, sa1_w1, sa1_b1,
           sa2_w0, sa2_b0, sa2_w1, sa2_b1, cls_w0, cls_b0, cls_w1, cls_b1,
           cls_w2, cls_b2, reg_w0, reg_b0, reg_w1, reg_b1, reg_w2, reg_b2):
    xyz = x[..., 0:3]
    xt0f = jnp.transpose(xyz, (2, 0, 1))
    xt0 = jnp.transpose(xyz, (0, 2, 1))

    new_xyz0 = _fps(xt0f, 128)
    feat0 = _sa(x, xt0, new_xyz0, sa0_w0, sa0_b0, sa0_w1, sa0_b1,
                0.2, 32)

    xt1f = jnp.transpose(new_xyz0, (2, 0, 1))
    xt1 = jnp.transpose(new_xyz0, (0, 2, 1))
    new_xyz1 = _fps(xt1f, 32)
    cat1 = jnp.concatenate([new_xyz0, feat0], axis=-1)
    feat1 = _sa(cat1, xt1, new_xyz1, sa1_w0, sa1_b0, sa1_w1, sa1_b1,
                0.4, 32)

    cat2 = jnp.concatenate([new_xyz1, feat1], axis=-1)
    pred_box, pred_class = _tail(
        cat2, sa2_w0, sa2_b0, sa2_w1, sa2_b1,
        cls_w0, cls_b0, cls_w1, cls_b1, cls_w2, cls_b2,
        reg_w0, reg_b0, reg_w1, reg_b1, reg_w2, reg_b2)
    return (pred_box, pred_class)


# speedup vs baseline: 1.0556x; 1.0556x over previous
import functools

import jax
import jax.numpy as jnp
from jax.experimental import pallas as pl

_F32 = jnp.float32
_BF16 = jnp.bfloat16


def _fps_body(xt_ref, out_ref, *, n, npoint):
    b = xt_ref.shape[1]
    x0 = xt_ref[0]
    x1 = xt_ref[1]
    x2 = xt_ref[2]
    iota = jax.lax.broadcasted_iota(jnp.int32, (b, n), 1)

    def body(i, state):
        dists, far = state
        onehot = iota == far
        c0 = jnp.sum(jnp.where(onehot, x0, 0.0), axis=1, keepdims=True)
        c1 = jnp.sum(jnp.where(onehot, x1, 0.0), axis=1, keepdims=True)
        c2 = jnp.sum(jnp.where(onehot, x2, 0.0), axis=1, keepdims=True)
        cent = jnp.concatenate([c0, c1, c2], axis=1).reshape(b, 1, 3)
        out_ref[:, pl.ds(i, 1), :] = cent
        d = (x0 - c0) ** 2 + (x1 - c1) ** 2 + (x2 - c2) ** 2
        dists = jnp.minimum(dists, d)
        m = jnp.max(dists, axis=1, keepdims=True)
        cand = jnp.where(dists == m, iota, n)
        far = jnp.min(cand, axis=1, keepdims=True)
        return dists, far

    dists = jnp.full((b, n), 1e10, _F32)
    far = jnp.zeros((b, 1), jnp.int32)
    jax.lax.fori_loop(0, npoint, body, (dists, far))


def _fps(xt, npoint):
    _, b, n = xt.shape
    return pl.pallas_call(
        functools.partial(_fps_body, n=n, npoint=npoint),
        out_shape=jax.ShapeDtypeStruct((b, npoint, 3), _F32),
    )(xt)


def _sa_body(cat_ref, xt_ref, nx_ref, w0t_ref, b0_ref, w1t_ref, b1_ref,
             out_ref, *, r2, nsample):
    n = cat_ref.shape[1]
    s = nx_ref.shape[1]
    cat = cat_ref[0]
    xt = xt_ref[0]
    nx = nx_ref[0]

    p = jnp.dot(cat, w0t_ref[...], preferred_element_type=_F32)
    off = jnp.dot(nx, w0t_ref[0:3, :], preferred_element_type=_F32)

    sqr = ((nx[:, 0:1] - xt[0:1, :]) ** 2
           + (nx[:, 1:2] - xt[1:2, :]) ** 2
           + (nx[:, 2:3] - xt[2:3, :]) ** 2)
    mask = (sqr < r2).astype(_F32)

    tri = (jax.lax.broadcasted_iota(jnp.int32, (n, n), 0)
           <= jax.lax.broadcasted_iota(jnp.int32, (n, n), 1)).astype(_BF16)
    rank = jnp.dot(mask.astype(_BF16), tri, preferred_element_type=_F32)
    count = rank[:, n - 1:n]
    vrank = jnp.where(mask > 0.5, rank, 0.0)

    b0 = b0_ref[...]
    b1 = b1_ref[...]
    w1t = w1t_ref[...].astype(_BF16)
    pb = p.astype(_BF16)
    cm = pb.shape[1]

    kvec = (jax.lax.broadcasted_iota(jnp.int32, (nsample, 1, 1), 0)
            .astype(_F32) + 1.0)
    sel = (vrank[None, :, :] == kvec).astype(_BF16)
    g = jnp.dot(sel.reshape(nsample * s, n), pb,
                preferred_element_type=_F32).reshape(nsample, s, cm)
    g0 = g[0]
    g = jnp.where(count[None, :, :] >= kvec, g, g0[None, :, :])
    a = jnp.maximum(g - off[None, :, :] + b0[None, :, :], 0.0)
    z = jnp.dot(a.astype(_BF16).reshape(nsample * s, cm), w1t,
                preferred_element_type=_F32)
    z = z.reshape(nsample, s, z.shape[-1]) + b1[None, :, :]
    out_ref[0] = jnp.maximum(jnp.max(z, axis=0), 0.0)


def _sa(cat, xt, nx, w0, b0, w1, b1, radius, nsample):
    b, n, c = cat.shape
    s = nx.shape[1]
    cm = w0.shape[0]
    co = w1.shape[0]
    grid = (b,)
    return pl.pallas_call(
        functools.partial(_sa_body, r2=radius * radius, nsample=nsample),
        grid=grid,
        in_specs=[
            pl.BlockSpec((1, n, c), lambda i: (i, 0, 0)),
            pl.BlockSpec((1, 3, n), lambda i: (i, 0, 0)),
            pl.BlockSpec((1, s, 3), lambda i: (i, 0, 0)),
            pl.BlockSpec((c, cm), lambda i: (0, 0)),
            pl.BlockSpec((1, cm), lambda i: (0, 0)),
            pl.BlockSpec((cm, co), lambda i: (0, 0)),
            pl.BlockSpec((1, co), lambda i: (0, 0)),
        ],
        out_specs=pl.BlockSpec((1, s, co), lambda i: (i, 0, 0)),
        out_shape=jax.ShapeDtypeStruct((b, s, co), _F32),
    )(cat, xt, nx, w0.T, b0[None, :], w1.T, b1[None, :])


def _tail_body(cat_ref, w20t_ref, b20_ref, w21t_ref, b21_ref,
               cw0t_ref, cb0_ref, cw1t_ref, cb1_ref, cw2t_ref, cb2_ref,
               rw0t_ref, rb0_ref, rw1t_ref, rb1_ref, rw2t_ref, rb2_ref,
               box_ref, cls_ref):
    b, ns, c = cat_ref.shape
    h = cat_ref[...].reshape(b * ns, c)
    h = jnp.maximum(jnp.dot(h, w20t_ref[...], preferred_element_type=_F32)
                    + b20_ref[...], 0.0)
    h = jnp.maximum(jnp.dot(h, w21t_ref[...], preferred_element_type=_F32)
                    + b21_ref[...], 0.0)
    g = jnp.max(h.reshape(b, ns, h.shape[-1]), axis=1)

    cc = jnp.maximum(jnp.dot(g, cw0t_ref[...], preferred_element_type=_F32)
                     + cb0_ref[...], 0.0)
    cc = jnp.maximum(jnp.dot(cc, cw1t_ref[...], preferred_element_type=_F32)
                     + cb1_ref[...], 0.0)
    cc = jnp.dot(cc, cw2t_ref[...], preferred_element_type=_F32) + cb2_ref[...]
    cls_ref[...] = jax.nn.sigmoid(cc)

    rr = jnp.maximum(jnp.dot(g, rw0t_ref[...], preferred_element_type=_F32)
                     + rb0_ref[...], 0.0)
    rr = jnp.maximum(jnp.dot(rr, rw1t_ref[...], preferred_element_type=_F32)
                     + rb1_ref[...], 0.0)
    box_ref[...] = (jnp.dot(rr, rw2t_ref[...], preferred_element_type=_F32)
                    + rb2_ref[...])


def _tail(cat2, w20, b20, w21, b21, cw0, cb0, cw1, cb1, cw2, cb2,
          rw0, rb0, rw1, rb1, rw2, rb2):
    b = cat2.shape[0]
    args = (cat2, w20.T, b20[None, :], w21.T, b21[None, :],
            cw0.T, cb0[None, :], cw1.T, cb1[None, :], cw2.T, cb2[None, :],
            rw0.T, rb0[None, :], rw1.T, rb1[None, :], rw2.T, rb2[None, :])
    return pl.pallas_call(
        _tail_body,
        out_shape=(jax.ShapeDtypeStruct((b, 7), _F32),
                   jax.ShapeDtypeStruct((b, 1), _F32)),
    )(*args)


def kernel(x, sa0_w0, sa0_b0, sa0_w1, sa0_b1, sa1_w0, sa1_b0, sa1_w1, sa1_b1,
           sa2_w0, sa2_b0, sa2_w1, sa2_b1, cls_w0, cls_b0, cls_w1, cls_b1,
           cls_w2, cls_b2, reg_w0, reg_b0, reg_w1, reg_b1, reg_w2, reg_b2):
    xyz = x[..., 0:3]
    xt0f = jnp.transpose(xyz, (2, 0, 1))
    xt0 = jnp.transpose(xyz, (0, 2, 1))

    new_xyz0 = _fps(xt0f, 128)
    feat0 = _sa(x, xt0, new_xyz0, sa0_w0, sa0_b0, sa0_w1, sa0_b1,
                0.2, 32)

    xt1f = jnp.transpose(new_xyz0, (2, 0, 1))
    xt1 = jnp.transpose(new_xyz0, (0, 2, 1))
    new_xyz1 = _fps(xt1f, 32)
    cat1 = jnp.concatenate([new_xyz0, feat0], axis=-1)
    feat1 = _sa(cat1, xt1, new_xyz1, sa1_w0, sa1_b0, sa1_w1, sa1_b1,
                0.4, 32)

    cat2 = jnp.concatenate([new_xyz1, feat1], axis=-1)
    pred_box, pred_class = _tail(
        cat2, sa2_w0, sa2_b0, sa2_w1, sa2_b1,
        cls_w0, cls_b0, cls_w1, cls_b1, cls_w2, cls_b2,
        reg_w0, reg_b0, reg_w1, reg_b1, reg_w2, reg_b2)
    return (pred_box, pred_class)


# speedup vs baseline: 1.1544x; 1.0935x over previous
import functools

import jax
import jax.numpy as jnp
from jax.experimental import pallas as pl

_F32 = jnp.float32
_BF16 = jnp.bfloat16


def _fps_body(xt_ref, out_ref, *, n, npoint):
    b = xt_ref.shape[1]
    x0 = xt_ref[0]
    x1 = xt_ref[1]
    x2 = xt_ref[2]
    iota = jax.lax.broadcasted_iota(jnp.int32, (b, n), 1)

    def body(i, state):
        dists, far = state
        onehot = iota == far
        c0 = jnp.sum(jnp.where(onehot, x0, 0.0), axis=1, keepdims=True)
        c1 = jnp.sum(jnp.where(onehot, x1, 0.0), axis=1, keepdims=True)
        c2 = jnp.sum(jnp.where(onehot, x2, 0.0), axis=1, keepdims=True)
        cent = jnp.concatenate([c0, c1, c2], axis=1).reshape(b, 1, 3)
        out_ref[:, pl.ds(i, 1), :] = cent
        d = (x0 - c0) ** 2 + (x1 - c1) ** 2 + (x2 - c2) ** 2
        dists = jnp.minimum(dists, d)
        m = jnp.max(dists, axis=1, keepdims=True)
        cand = jnp.where(dists == m, iota, n)
        far = jnp.min(cand, axis=1, keepdims=True)
        return dists, far

    dists = jnp.full((b, n), 1e10, _F32)
    far = jnp.zeros((b, 1), jnp.int32)
    jax.lax.fori_loop(0, npoint, body, (dists, far))


def _fps(xt, npoint):
    _, b, n = xt.shape
    return pl.pallas_call(
        functools.partial(_fps_body, n=n, npoint=npoint),
        out_shape=jax.ShapeDtypeStruct((b, npoint, 3), _F32),
    )(xt)


def _sa_body(cat_ref, xt_ref, nx_ref, w0t_ref, b0_ref, w1t_ref, b1_ref,
             out_ref, *, r2, nsample):
    tri = None
    for t in range(cat_ref.shape[0]):
        tri = _sa_one(cat_ref[t], xt_ref[t], nx_ref[t], w0t_ref, b0_ref,
                      w1t_ref, b1_ref, out_ref, t, tri, r2=r2,
                      nsample=nsample)


def _sa_one(cat, xt, nx, w0t_ref, b0_ref, w1t_ref, b1_ref, out_ref, t, tri,
            *, r2, nsample):
    n = cat.shape[0]
    s = nx.shape[0]

    p = jnp.dot(cat, w0t_ref[...], preferred_element_type=_F32)
    off = jnp.dot(nx, w0t_ref[0:3, :], preferred_element_type=_F32)

    sqr = ((nx[:, 0:1] - xt[0:1, :]) ** 2
           + (nx[:, 1:2] - xt[1:2, :]) ** 2
           + (nx[:, 2:3] - xt[2:3, :]) ** 2)
    mask = (sqr < r2).astype(_F32)

    if tri is None:
        tri = (jax.lax.broadcasted_iota(jnp.int32, (n, n), 0)
               <= jax.lax.broadcasted_iota(jnp.int32, (n, n), 1)).astype(_BF16)
    rank = jnp.dot(mask.astype(_BF16), tri, preferred_element_type=_F32)
    count = rank[:, n - 1:n]
    vrank = jnp.where(mask > 0.5, rank, 0.0)

    b0 = b0_ref[...]
    b1 = b1_ref[...]
    w1t = w1t_ref[...].astype(_BF16)
    pb = p.astype(_BF16)
    cm = pb.shape[1]

    kvec = (jax.lax.broadcasted_iota(jnp.int32, (nsample, 1, 1), 0)
            .astype(_F32) + 1.0)
    sel = (vrank[None, :, :] == kvec).astype(_BF16)
    g = jnp.dot(sel.reshape(nsample * s, n), pb,
                preferred_element_type=_F32).reshape(nsample, s, cm)
    g0 = g[0]
    g = jnp.where(count[None, :, :] >= kvec, g, g0[None, :, :])
    a = jnp.maximum(g - off[None, :, :] + b0[None, :, :], 0.0)
    z = jnp.dot(a.astype(_BF16).reshape(nsample * s, cm), w1t,
                preferred_element_type=_F32)
    z = z.reshape(nsample, s, z.shape[-1]) + b1[None, :, :]
    out_ref[t] = jnp.maximum(jnp.max(z, axis=0), 0.0)
    return tri


def _sa(cat, xt, nx, w0, b0, w1, b1, radius, nsample, bpg=1):
    b, n, c = cat.shape
    s = nx.shape[1]
    cm = w0.shape[0]
    co = w1.shape[0]
    grid = (b // bpg,)
    return pl.pallas_call(
        functools.partial(_sa_body, r2=radius * radius, nsample=nsample),
        grid=grid,
        in_specs=[
            pl.BlockSpec((bpg, n, c), lambda i: (i, 0, 0)),
            pl.BlockSpec((bpg, 3, n), lambda i: (i, 0, 0)),
            pl.BlockSpec((bpg, s, 3), lambda i: (i, 0, 0)),
            pl.BlockSpec((c, cm), lambda i: (0, 0)),
            pl.BlockSpec((1, cm), lambda i: (0, 0)),
            pl.BlockSpec((cm, co), lambda i: (0, 0)),
            pl.BlockSpec((1, co), lambda i: (0, 0)),
        ],
        out_specs=pl.BlockSpec((bpg, s, co), lambda i: (i, 0, 0)),
        out_shape=jax.ShapeDtypeStruct((b, s, co), _F32),
    )(cat, xt, nx, w0.T, b0[None, :], w1.T, b1[None, :])


def _tail_body(cat_ref, w20t_ref, b20_ref, w21t_ref, b21_ref,
               cw0t_ref, cb0_ref, cw1t_ref, cb1_ref, cw2t_ref, cb2_ref,
               rw0t_ref, rb0_ref, rw1t_ref, rb1_ref, rw2t_ref, rb2_ref,
               box_ref, cls_ref):
    b, ns, c = cat_ref.shape
    h = cat_ref[...].reshape(b * ns, c)
    h = jnp.maximum(jnp.dot(h, w20t_ref[...], preferred_element_type=_F32)
                    + b20_ref[...], 0.0)
    h = jnp.maximum(jnp.dot(h, w21t_ref[...], preferred_element_type=_F32)
                    + b21_ref[...], 0.0)
    g = jnp.max(h.reshape(b, ns, h.shape[-1]), axis=1)

    cc = jnp.maximum(jnp.dot(g, cw0t_ref[...], preferred_element_type=_F32)
                     + cb0_ref[...], 0.0)
    cc = jnp.maximum(jnp.dot(cc, cw1t_ref[...], preferred_element_type=_F32)
                     + cb1_ref[...], 0.0)
    cc = jnp.dot(cc, cw2t_ref[...], preferred_element_type=_F32) + cb2_ref[...]
    cls_ref[...] = jax.nn.sigmoid(cc)

    rr = jnp.maximum(jnp.dot(g, rw0t_ref[...], preferred_element_type=_F32)
                     + rb0_ref[...], 0.0)
    rr = jnp.maximum(jnp.dot(rr, rw1t_ref[...], preferred_element_type=_F32)
                     + rb1_ref[...], 0.0)
    box_ref[...] = (jnp.dot(rr, rw2t_ref[...], preferred_element_type=_F32)
                    + rb2_ref[...])


def _tail(cat2, w20, b20, w21, b21, cw0, cb0, cw1, cb1, cw2, cb2,
          rw0, rb0, rw1, rb1, rw2, rb2):
    b = cat2.shape[0]
    args = (cat2, w20.T, b20[None, :], w21.T, b21[None, :],
            cw0.T, cb0[None, :], cw1.T, cb1[None, :], cw2.T, cb2[None, :],
            rw0.T, rb0[None, :], rw1.T, rb1[None, :], rw2.T, rb2[None, :])
    return pl.pallas_call(
        _tail_body,
        out_shape=(jax.ShapeDtypeStruct((b, 7), _F32),
                   jax.ShapeDtypeStruct((b, 1), _F32)),
    )(*args)


def kernel(x, sa0_w0, sa0_b0, sa0_w1, sa0_b1, sa1_w0, sa1_b0, sa1_w1, sa1_b1,
           sa2_w0, sa2_b0, sa2_w1, sa2_b1, cls_w0, cls_b0, cls_w1, cls_b1,
           cls_w2, cls_b2, reg_w0, reg_b0, reg_w1, reg_b1, reg_w2, reg_b2):
    xyz = x[..., 0:3]
    xt0f = jnp.transpose(xyz, (2, 0, 1))
    xt0 = jnp.transpose(xyz, (0, 2, 1))

    new_xyz0 = _fps(xt0f, 128)
    feat0 = _sa(x, xt0, new_xyz0, sa0_w0, sa0_b0, sa0_w1, sa0_b1,
                0.2, 32)

    xt1f = jnp.transpose(new_xyz0, (2, 0, 1))
    xt1 = jnp.transpose(new_xyz0, (0, 2, 1))
    new_xyz1 = _fps(xt1f, 32)
    cat1 = jnp.concatenate([new_xyz0, feat0], axis=-1)
    feat1 = _sa(cat1, xt1, new_xyz1, sa1_w0, sa1_b0, sa1_w1, sa1_b1,
                0.4, 32, bpg=8)

    cat2 = jnp.concatenate([new_xyz1, feat1], axis=-1)
    pred_box, pred_class = _tail(
        cat2, sa2_w0, sa2_b0, sa2_w1, sa2_b1,
        cls_w0, cls_b0, cls_w1, cls_b1, cls_w2, cls_b2,
        reg_w0, reg_b0, reg_w1, reg_b1, reg_w2, reg_b2)
    return (pred_box, pred_class)
